# baseline (speedup 1.0000x reference)
import jax, jax.numpy as jnp
from jax import lax
from jax.experimental import pallas as pl
from jax.experimental.pallas import tpu as pltpu, tpu_sc as plsc

NC, NS = 2, 16
NW = NC * NS
DEPTH = 8


def body(ids_hbm, table_hbm, out_hbm, idx_all, tab_v, sem):
    wid = lax.axis_index("s") * NC + lax.axis_index("c")
    n = ids_hbm.shape[0]
    pw = n // NW
    base = wid * pw
    ngroups = pw // 16

    pltpu.sync_copy(ids_hbm.at[pl.ds(base, pw)], idx_all)
    pltpu.sync_copy(table_hbm, tab_v)

    def issue_group(g):
        ids_v = idx_all[pl.ds(g * 16, 16)]
        goff = base + g * 16
        for l in range(16):
            tid = ids_v[l]
            pltpu.async_copy(tab_v.at[pl.ds(tid, 1)],
                             out_hbm.at[pl.ds(goff + l, 1)], sem)

    def drain_group():
        pltpu.make_async_copy(tab_v.at[pl.ds(0, 16)],
                              out_hbm.at[pl.ds(0, 16)], sem).wait()

    @pl.loop(0, DEPTH)
    def _prime(g):
        issue_group(g)

    @pl.loop(DEPTH, ngroups)
    def _steady(g):
        drain_group()
        issue_group(g)

    @pl.loop(0, DEPTH)
    def _tail(g):
        drain_group()


def kernel(token_ids, table):
    b, s = token_ids.shape
    v, d = table.shape
    ids = token_ids.reshape(-1).astype(jnp.int32)
    n = ids.shape[0]
    mesh = plsc.VectorSubcoreMesh(core_axis_name="c", subcore_axis_name="s",
                                  num_cores=NC, num_subcores=NS)
    out = pl.kernel(
        body, out_type=jax.ShapeDtypeStruct((n, d), jnp.float32), mesh=mesh,
        compiler_params=pltpu.CompilerParams(needs_layout_passes=False),
        scratch_types=[
            pltpu.VMEM((n // NW,), jnp.int32),
            pltpu.VMEM((v, d), jnp.float32),
            pltpu.SemaphoreType.DMA,
        ],
    )(ids, table)
    return out.reshape(b, s, d)
